# Initial kernel scaffold; baseline (speedup 1.0000x reference)
import jax
import jax.numpy as jnp
from jax.experimental import pallas as pl


def kernel(x, edge_index, edge_attr, batch, emb, W1, a_s1, a_d1, We1, ae1, b1, W2, a_s2, a_d2, We2, ae2, b2, W3, a_s3, a_d3, We3, ae3, b3):
    raise NotImplementedError("write your pallas kernel here")


# speedup vs baseline: 4.8402x; 4.8402x over previous
import functools
import jax
import jax.numpy as jnp
from jax import lax
from jax.experimental import pallas as pl
from jax.experimental.pallas import tpu as pltpu
from jax.experimental.pallas import tpu_sc as plsc

N = 10000
E = 320000
F_IN = 128
EMB = 50
H = 256
C = 40
DE = 16
G = 64

N_PAD = 10240
E_PAD = 327680
N_HALF = 5120
SPLIT = 5000
F3 = 128
NEG = -1e30


def _edge_feat_body(ea2_ref, bd_ref, t16_ref, v8_ref, c_ref, clv_ref):
    ea2 = ea2_ref[...]
    pad = jnp.zeros((E_PAD // 8 - E // 8, 128), jnp.float32)
    c_ref[...] = jnp.concatenate([ea2, pad], axis=0) @ bd_ref[...]
    s128 = jnp.sum(ea2, axis=0, keepdims=True)
    clv_ref[...] = ((s128 @ t16_ref[...]) / E) @ v8_ref[...]


def _edge_feat(ea2, bd, t16, v8):
    return pl.pallas_call(
        _edge_feat_body,
        out_shape=(
            jax.ShapeDtypeStruct((E_PAD // 8, 64), jnp.float32),
            jax.ShapeDtypeStruct((1, 8), jnp.float32),
        ),
    )(ea2, bd, t16, v8)


def _attn_cols(h, a_s, a_d, clv, li):
    asc = h @ a_s[:, None]
    adc = h @ a_d[:, None]
    al = asc + adc + clv[0, li]
    al = jnp.where(al >= 0, al, 0.2 * al)
    top = jnp.concatenate([asc, adc, al], 1)
    z = jnp.zeros((N_PAD - N, 2), jnp.float32)
    negs = jnp.full((N_PAD - N, 1), NEG, jnp.float32)
    return jnp.concatenate([top, jnp.concatenate([z, negs], 1)], 0)


def _embed_body(x_ref, emb_ref, w_ref, as_ref, ad_ref, clv_ref,
                h_ref, pack_ref):
    x = x_ref[...]
    rm = jnp.max(x, axis=1, keepdims=True)
    ii = lax.broadcasted_iota(jnp.int32, x.shape, 1)
    cand = jnp.where(x == rm, ii, F_IN)
    idx = jnp.min(cand, axis=1, keepdims=True)
    oh = (lax.broadcasted_iota(jnp.int32, (N, F_IN), 1) == idx
          ).astype(jnp.float32)
    h0 = oh @ emb_ref[...]
    h = h0 @ w_ref[...]
    h_ref[...] = h
    pack_ref[...] = _attn_cols(h, as_ref[...], ad_ref[...], clv_ref[...], 0)


def _embed_l1(x, emb, W1, a_s1, a_d1, clv):
    return pl.pallas_call(
        _embed_body,
        out_shape=(
            jax.ShapeDtypeStruct((N, H), jnp.float32),
            jax.ShapeDtypeStruct((N_PAD, 3), jnp.float32),
        ),
    )(x, emb, W1, a_s1, a_d1, clv)


def _gmax_body(tmax_ref, al_ref, m_ref):
    m = jnp.maximum(jnp.max(tmax_ref[...]), jnp.max(al_ref[...]))
    m_ref[...] = jnp.full((1, 16), m, jnp.float32)


def _gmax(tmax, al_loop):
    return pl.pallas_call(
        _gmax_body,
        out_shape=jax.ShapeDtypeStruct((1, 16), jnp.float32),
    )(tmax, al_loop)


def _combine(numer_ref, denp_ref, al_ref, m_ref, h_ref, b_ref):
    m = m_ref[0, 0]
    exl = jnp.exp(al_ref[...][:N] - m)
    den = denp_ref[:N, 0:1]
    nm = numer_ref[:N]
    h = h_ref[...]
    return (nm + exl * h) / (den + exl) + b_ref[...][None, :]


def _post_body(numer_ref, denp_ref, al_ref, m_ref, h_ref, b_ref,
               w_ref, as_ref, ad_ref, clv_ref,
               h2_ref, pack_ref, *, li):
    out = _combine(numer_ref, denp_ref, al_ref, m_ref, h_ref, b_ref)
    out = jnp.maximum(out, 0.0)
    h2 = out @ w_ref[...]
    h2_ref[...] = h2
    pack_ref[...] = _attn_cols(h2, as_ref[...], ad_ref[...], clv_ref[...], li)


def _post_layer(numer, denp, al_loop, m, h, b, w_next, as_next, ad_next,
                clv, li, f_next):
    return pl.pallas_call(
        functools.partial(_post_body, li=li),
        out_shape=(
            jax.ShapeDtypeStruct((N, f_next), jnp.float32),
            jax.ShapeDtypeStruct((N_PAD, 3), jnp.float32),
        ),
    )(numer, denp, al_loop, m, h, b, w_next, as_next, ad_next, clv)


def _final_body(numer_ref, denp_ref, al_ref, m_ref, h_ref, b_ref,
                bat_ref, out_ref):
    out3 = _combine(numer_ref, denp_ref, al_ref, m_ref, h_ref, b_ref)[:, :C]
    oh = (bat_ref[...] == lax.broadcasted_iota(jnp.int32, (G, N), 0)
          ).astype(jnp.float32)
    sums = oh @ out3
    cnt = oh @ jnp.ones((N, 1), jnp.float32)
    pooled = sums / jnp.maximum(cnt, 1.0)
    pm = jnp.max(pooled, axis=1, keepdims=True)
    ex = jnp.exp(pooled - pm)
    out_ref[...] = ex / jnp.sum(ex, axis=1, keepdims=True)


def _final(numer, denp, al_loop, m, h, b3p, bat_row):
    return pl.pallas_call(
        _final_body,
        out_shape=jax.ShapeDtypeStruct((G, C), jnp.float32),
    )(numer, denp, al_loop, m, h, b3p, bat_row)


NW = 32
EPT = E_PAD // NW
ROWS_PT = E_PAD // 128 // NW
CHR = 16
RG = 64

_sc_mesh = plsc.VectorSubcoreMesh(core_axis_name="c", subcore_axis_name="s")
_sc_params = pltpu.CompilerParams(needs_layout_passes=False)


def _sc_alpha_body(asc_hbm, adc_hbm, src_hbm, dst_hbm, c_hbm,
                   alpha_hbm, tmax_hbm,
                   as_v, ad_v, src_v, dst_v, c_v, alpha_v, tm_v):
    cid = lax.axis_index("c")
    sid = lax.axis_index("s")
    wid = sid * 2 + cid
    rbase = wid * ROWS_PT
    pltpu.sync_copy(asc_hbm, as_v)
    pltpu.sync_copy(adc_hbm, ad_v)
    tm_v[...] = jnp.full((16,), NEG, jnp.float32)

    def chunk(k, _):
        roff = rbase + k * CHR
        pltpu.sync_copy(src_hbm.at[pl.ds(roff, CHR)], src_v)
        pltpu.sync_copy(dst_hbm.at[pl.ds(roff, CHR)], dst_v)
        pltpu.sync_copy(c_hbm.at[pl.ds(roff, CHR)], c_v)

        def vec(j, _):
            r = j // 8
            q = (j % 8) * 16
            s = src_v[r, pl.ds(q, 16)]
            d = dst_v[r, pl.ds(q, 16)]
            a = (plsc.load_gather(as_v, [s]) + plsc.load_gather(ad_v, [d])
                 + c_v[r, pl.ds(q, 16)])
            a = jnp.where(a >= 0, a, 0.2 * a)
            alpha_v[r, pl.ds(q, 16)] = a
            tm_v[...] = jnp.maximum(tm_v[...], a)
            return 0

        lax.fori_loop(0, CHR * 8, vec, 0)
        pltpu.sync_copy(alpha_v, alpha_hbm.at[pl.ds(roff, CHR)])
        return 0

    lax.fori_loop(0, ROWS_PT // CHR, chunk, 0)
    pltpu.sync_copy(tm_v, tmax_hbm.at[wid])


def _sc_alpha(asc, adc, src2d, dst2d, c_l):
    fn = pl.kernel(
        _sc_alpha_body,
        out_type=(jax.ShapeDtypeStruct((E_PAD // 128, 128), jnp.float32),
                  jax.ShapeDtypeStruct((NW, 16), jnp.float32)),
        mesh=_sc_mesh,
        compiler_params=_sc_params,
        scratch_types=[
            pltpu.VMEM((N_PAD,), jnp.float32),
            pltpu.VMEM((N_PAD,), jnp.float32),
            pltpu.VMEM((CHR, 128), jnp.int32),
            pltpu.VMEM((CHR, 128), jnp.int32),
            pltpu.VMEM((CHR, 128), jnp.float32),
            pltpu.VMEM((CHR, 128), jnp.float32),
            pltpu.VMEM((16,), jnp.float32),
        ],
    )
    return fn(asc, adc, src2d, dst2d, c_l)


STRIPE = N_PAD // NW
NCHUNK = E_PAD // 2048


def _sc_scatter_body(alpha_hbm, m_hbm, src_hbm, dst_hbm, h_hbm,
                     numer_hbm, denp_hbm,
                     acc, accd, src_v, dst_v, alpha_v, sbuf, dbuf, ebuf,
                     rows_v, m_v, gsem, *, f):
    cid = lax.axis_index("c")
    sid = lax.axis_index("s")
    wid = sid * 2 + cid
    sb = wid * STRIPE
    nq = f // 16
    zf = jnp.zeros((16,), jnp.float32)
    zi = jnp.zeros((16,), jnp.int32)

    def zacc(r, _):
        acc[pl.ds(r * 16, 16)] = zf
        return 0

    lax.fori_loop(0, STRIPE * f // 16, zacc, 0)

    def zaccd(r, _):
        accd[pl.ds(r * 16, 16)] = zf
        return 0

    lax.fori_loop(0, STRIPE, zaccd, 0)

    def zbuf(r, _):
        sbuf[pl.ds(r * 16, 16)] = zi
        dbuf[pl.ds(r * 16, 16)] = zi
        ebuf[pl.ds(r * 16, 16)] = zf
        return 0

    lax.fori_loop(0, 128, zbuf, 0)
    pltpu.sync_copy(m_hbm, m_v)
    mvec = m_v[...]
    iota16 = lax.broadcasted_iota(jnp.int32, (16,), 0)

    def chunk(k, _):
        roff = k * CHR
        pltpu.sync_copy(src_hbm.at[pl.ds(roff, CHR)], src_v)
        pltpu.sync_copy(dst_hbm.at[pl.ds(roff, CHR)], dst_v)
        pltpu.sync_copy(alpha_hbm.at[pl.ds(roff, CHR)], alpha_v)

        def vec(j, off):
            s = src_v[j // 8, pl.ds((j % 8) * 16, 16)]
            d = dst_v[j // 8, pl.ds((j % 8) * 16, 16)]
            a = alpha_v[j // 8, pl.ds((j % 8) * 16, 16)]
            e = jnp.exp(a - mvec)
            dl = d - sb
            mask = (dl >= 0) & (dl < STRIPE)
            cs = plsc.cumsum(mask.astype(jnp.int32))
            p = off + cs - 1
            plsc.store_scatter(sbuf, [p], s, mask=mask)
            plsc.store_scatter(dbuf, [p], dl, mask=mask)
            plsc.store_scatter(ebuf, [p], e, mask=mask)
            return off + jnp.max(cs)

        off = lax.fori_loop(0, CHR * 8, vec, 0)
        ng = (off + 15) // 16

        def agg(g, _):
            pltpu.async_copy(h_hbm.at[sbuf.at[pl.ds(g * 16, 16)]],
                             rows_v, gsem).wait()
            lane = g * 16 + iota16
            emv = ebuf[pl.ds(g * 16, 16)]
            emv = jnp.where(lane < off, emv, 0.0)
            ebuf[pl.ds(g * 16, 16)] = emv

            def row(r, _):
                gi = jnp.full((16,), g * 16 + r, jnp.int32)
                em_b = plsc.load_gather(ebuf, [gi])
                dlv = plsc.load_gather(dbuf, [gi])
                base = dlv * f + iota16
                for q in range(nq):
                    plsc.addupdate_scatter(
                        acc, [base + q * 16],
                        rows_v[r, pl.ds(q * 16, 16)] * em_b)
                plsc.addupdate_scatter(accd, [dlv * 16 + iota16], em_b)
                return 0

            lax.fori_loop(0, 16, row, 0)
            return 0

        lax.fori_loop(0, ng, agg, 0)
        return 0

    lax.fori_loop(0, NCHUNK, chunk, 0)
    pltpu.sync_copy(acc, numer_hbm.at[pl.ds(sb * f, STRIPE * f)])
    pltpu.sync_copy(accd, denp_hbm.at[pl.ds(sb * 16, STRIPE * 16)])


def _sc_scatter(alpha2d, m, src2d, dst2d, h, f):
    fn = pl.kernel(
        functools.partial(_sc_scatter_body, f=f),
        out_type=(jax.ShapeDtypeStruct((N_PAD * f,), jnp.float32),
                  jax.ShapeDtypeStruct((N_PAD * 16,), jnp.float32)),
        mesh=_sc_mesh,
        compiler_params=_sc_params,
        scratch_types=[
            pltpu.VMEM((STRIPE * f,), jnp.float32),
            pltpu.VMEM((STRIPE * 16,), jnp.float32),
            pltpu.VMEM((CHR, 128), jnp.int32),
            pltpu.VMEM((CHR, 128), jnp.int32),
            pltpu.VMEM((CHR, 128), jnp.float32),
            pltpu.VMEM((CHR * 128,), jnp.int32),
            pltpu.VMEM((CHR * 128,), jnp.int32),
            pltpu.VMEM((CHR * 128,), jnp.float32),
            pltpu.VMEM((16, f), jnp.float32),
            pltpu.VMEM((16,), jnp.float32),
            pltpu.SemaphoreType.DMA,
        ],
    )
    return fn(alpha2d, m, src2d, dst2d, h)


def kernel(x, edge_index, edge_attr, batch, emb,
           W1, a_s1, a_d1, We1, ae1, b1,
           W2, a_s2, a_d2, We2, ae2, b2,
           W3, a_s3, a_d3, We3, ae3, b3):
    src = jnp.concatenate([edge_index[0],
                           jnp.zeros((E_PAD - E,), jnp.int32)])
    dst = jnp.concatenate([edge_index[1],
                           jnp.full((E_PAD - E,), N + 100, jnp.int32)])

    v8 = jnp.zeros((DE, 8), jnp.float32)
    v8 = v8.at[:, 0].set(We1 @ ae1).at[:, 1].set(We2 @ ae2)
    v8 = v8.at[:, 2].set(We3 @ ae3)
    ea2 = edge_attr.reshape(E // 8, 128)
    bd = jnp.kron(jnp.eye(8, dtype=jnp.float32), v8)
    t16 = jnp.kron(jnp.ones((8, 1), jnp.float32),
                   jnp.eye(16, dtype=jnp.float32))
    c64, clv = _edge_feat(ea2, bd, t16, v8)
    c8 = c64.reshape(E_PAD, 8)

    W3p = jnp.concatenate([W3, jnp.zeros((H, F3 - C), jnp.float32)], 1)
    b3p = jnp.concatenate([b3, jnp.zeros((F3 - C,), jnp.float32)])
    zc = jnp.zeros((F3 - C,), jnp.float32)
    a_s3p = jnp.concatenate([a_s3, zc])
    a_d3p = jnp.concatenate([a_d3, zc])

    h1, pack = _embed_l1(x, emb, W1, a_s1, a_d1, clv)

    src2d = src.reshape(E_PAD // 128, 128)
    dst2d = dst.reshape(E_PAD // 128, 128)

    layer_w = [(W2, a_s2, a_d2, b1, 1, H), (W3p, a_s3p, a_d3p, b2, 2, F3)]
    h, f = h1, H
    for i in range(3):
        c_l = c8[:, i].reshape(E_PAD // 128, 128)
        asc, adc, al = pack[:, 0], pack[:, 1], pack[:, 2:3]
        alpha2d, tmax = _sc_alpha(asc, adc, src2d, dst2d, c_l)
        m = _gmax(tmax, al)
        m16 = m.reshape(16)
        numer, denp = _sc_scatter(alpha2d, m16, src2d, dst2d, h, f)
        numer = numer.reshape(N_PAD, f)
        denp = denp.reshape(N_PAD, 16)
        if i < 2:
            w_n, as_n, ad_n, b_l, li, f_n = layer_w[i]
            h, pack = _post_layer(numer, denp, al, m, h, b_l,
                                  w_n, as_n, ad_n, clv, li, f_n)
            f = f_n
        else:
            bat_row = batch[None, :]
            out = _final(numer, denp, al, m, h, b3p, bat_row)
    return out


# speedup vs baseline: 6.2566x; 1.2926x over previous
import functools
import jax
import jax.numpy as jnp
from jax import lax
from jax.experimental import pallas as pl
from jax.experimental.pallas import tpu as pltpu
from jax.experimental.pallas import tpu_sc as plsc

N = 10000
E = 320000
F_IN = 128
EMB = 50
H = 256
C = 40
DE = 16
G = 64

N_PAD = 10240
E_PAD = 327680
N_HALF = 5120
SPLIT = 5000
F3 = 128
NEG = -1e30


def _edge_feat_body(ea2_ref, bd_ref, t16_ref, v8_ref, c_ref, clv_ref):
    ea2 = ea2_ref[...]
    pad = jnp.zeros((E_PAD // 8 - E // 8, 128), jnp.float32)
    c_ref[...] = jnp.concatenate([ea2, pad], axis=0) @ bd_ref[...]
    s128 = jnp.sum(ea2, axis=0, keepdims=True)
    clv_ref[...] = ((s128 @ t16_ref[...]) / E) @ v8_ref[...]


def _edge_feat(ea2, bd, t16, v8):
    return pl.pallas_call(
        _edge_feat_body,
        out_shape=(
            jax.ShapeDtypeStruct((E_PAD // 8, 64), jnp.float32),
            jax.ShapeDtypeStruct((1, 8), jnp.float32),
        ),
    )(ea2, bd, t16, v8)


def _attn_cols(h, a_s, a_d, clv, li):
    asc = h @ a_s[:, None]
    adc = h @ a_d[:, None]
    al = asc + adc + clv[0, li]
    al = jnp.where(al >= 0, al, 0.2 * al)
    top = jnp.concatenate([asc, adc, al], 1)
    z = jnp.zeros((N_PAD - N, 2), jnp.float32)
    negs = jnp.full((N_PAD - N, 1), NEG, jnp.float32)
    return jnp.concatenate([top, jnp.concatenate([z, negs], 1)], 0)


def _embed_body(x_ref, emb_ref, w_ref, as_ref, ad_ref, clv_ref,
                h_ref, pack_ref):
    x = x_ref[...]
    rm = jnp.max(x, axis=1, keepdims=True)
    ii = lax.broadcasted_iota(jnp.int32, x.shape, 1)
    cand = jnp.where(x == rm, ii, F_IN)
    idx = jnp.min(cand, axis=1, keepdims=True)
    oh = (lax.broadcasted_iota(jnp.int32, (N, F_IN), 1) == idx
          ).astype(jnp.float32)
    h0 = oh @ emb_ref[...]
    h = h0 @ w_ref[...]
    h_ref[...] = h
    pack_ref[...] = _attn_cols(h, as_ref[...], ad_ref[...], clv_ref[...], 0)


def _embed_l1(x, emb, W1, a_s1, a_d1, clv):
    return pl.pallas_call(
        _embed_body,
        out_shape=(
            jax.ShapeDtypeStruct((N, H), jnp.float32),
            jax.ShapeDtypeStruct((N_PAD, 3), jnp.float32),
        ),
    )(x, emb, W1, a_s1, a_d1, clv)


def _gmax_body(tmax_ref, al_ref, m_ref):
    m = jnp.maximum(jnp.max(tmax_ref[...]), jnp.max(al_ref[...]))
    m_ref[...] = jnp.full((1, 16), m, jnp.float32)


def _gmax(tmax, al_loop):
    return pl.pallas_call(
        _gmax_body,
        out_shape=jax.ShapeDtypeStruct((1, 16), jnp.float32),
    )(tmax, al_loop)


def _combine(numer_ref, denp_ref, al_ref, m_ref, h_ref, b_ref):
    m = m_ref[0, 0]
    exl = jnp.exp(al_ref[...][:N] - m)
    den = denp_ref[:N, 0:1]
    nm = numer_ref[:N]
    h = h_ref[...]
    return (nm + exl * h) / (den + exl) + b_ref[...][None, :]


def _post_body(numer_ref, denp_ref, al_ref, m_ref, h_ref, b_ref,
               w_ref, as_ref, ad_ref, clv_ref,
               h2_ref, pack_ref, *, li):
    out = _combine(numer_ref, denp_ref, al_ref, m_ref, h_ref, b_ref)
    out = jnp.maximum(out, 0.0)
    h2 = out @ w_ref[...]
    h2_ref[...] = h2
    pack_ref[...] = _attn_cols(h2, as_ref[...], ad_ref[...], clv_ref[...], li)


def _post_layer(numer, denp, al_loop, m, h, b, w_next, as_next, ad_next,
                clv, li, f_next):
    return pl.pallas_call(
        functools.partial(_post_body, li=li),
        out_shape=(
            jax.ShapeDtypeStruct((N, f_next), jnp.float32),
            jax.ShapeDtypeStruct((N_PAD, 3), jnp.float32),
        ),
    )(numer, denp, al_loop, m, h, b, w_next, as_next, ad_next, clv)


def _final_body(numer_ref, denp_ref, al_ref, m_ref, h_ref, b_ref,
                bat_ref, out_ref):
    out3 = _combine(numer_ref, denp_ref, al_ref, m_ref, h_ref, b_ref)[:, :C]
    oh = (bat_ref[...] == lax.broadcasted_iota(jnp.int32, (G, N), 0)
          ).astype(jnp.float32)
    sums = oh @ out3
    cnt = oh @ jnp.ones((N, 1), jnp.float32)
    pooled = sums / jnp.maximum(cnt, 1.0)
    pm = jnp.max(pooled, axis=1, keepdims=True)
    ex = jnp.exp(pooled - pm)
    out_ref[...] = ex / jnp.sum(ex, axis=1, keepdims=True)


def _final(numer, denp, al_loop, m, h, b3p, bat_row):
    return pl.pallas_call(
        _final_body,
        out_shape=jax.ShapeDtypeStruct((G, C), jnp.float32),
    )(numer, denp, al_loop, m, h, b3p, bat_row)


NW = 32
EPT = E_PAD // NW
ROWS_PT = E_PAD // 128 // NW
CHR = 16
RG = 64

_sc_mesh = plsc.VectorSubcoreMesh(core_axis_name="c", subcore_axis_name="s")
_sc_params = pltpu.CompilerParams(needs_layout_passes=False)


def _sc_alpha_body(asc_hbm, adc_hbm, src_hbm, dst_hbm, c_hbm,
                   alpha_hbm, tmax_hbm,
                   as_v, ad_v, src_v, dst_v, c_v, alpha_v, tm_v):
    cid = lax.axis_index("c")
    sid = lax.axis_index("s")
    wid = sid * 2 + cid
    rbase = wid * ROWS_PT
    pltpu.sync_copy(asc_hbm, as_v)
    pltpu.sync_copy(adc_hbm, ad_v)
    tm_v[...] = jnp.full((16,), NEG, jnp.float32)

    def chunk(k, _):
        roff = rbase + k * CHR
        pltpu.sync_copy(src_hbm.at[pl.ds(roff, CHR)], src_v)
        pltpu.sync_copy(dst_hbm.at[pl.ds(roff, CHR)], dst_v)
        pltpu.sync_copy(c_hbm.at[pl.ds(roff, CHR)], c_v)

        def vec(j, _):
            r = j // 8
            q = (j % 8) * 16
            s = src_v[r, pl.ds(q, 16)]
            d = dst_v[r, pl.ds(q, 16)]
            a = (plsc.load_gather(as_v, [s]) + plsc.load_gather(ad_v, [d])
                 + c_v[r, pl.ds(q, 16)])
            a = jnp.where(a >= 0, a, 0.2 * a)
            alpha_v[r, pl.ds(q, 16)] = a
            tm_v[...] = jnp.maximum(tm_v[...], a)
            return 0

        lax.fori_loop(0, CHR * 8, vec, 0)
        pltpu.sync_copy(alpha_v, alpha_hbm.at[pl.ds(roff, CHR)])
        return 0

    lax.fori_loop(0, ROWS_PT // CHR, chunk, 0)
    pltpu.sync_copy(tm_v, tmax_hbm.at[wid])


def _sc_alpha(asc, adc, src2d, dst2d, c_l):
    fn = pl.kernel(
        _sc_alpha_body,
        out_type=(jax.ShapeDtypeStruct((E_PAD // 128, 128), jnp.float32),
                  jax.ShapeDtypeStruct((NW, 16), jnp.float32)),
        mesh=_sc_mesh,
        compiler_params=_sc_params,
        scratch_types=[
            pltpu.VMEM((N_PAD,), jnp.float32),
            pltpu.VMEM((N_PAD,), jnp.float32),
            pltpu.VMEM((CHR, 128), jnp.int32),
            pltpu.VMEM((CHR, 128), jnp.int32),
            pltpu.VMEM((CHR, 128), jnp.float32),
            pltpu.VMEM((CHR, 128), jnp.float32),
            pltpu.VMEM((16,), jnp.float32),
        ],
    )
    return fn(asc, adc, src2d, dst2d, c_l)


STRIPE = N_PAD // NW
NCHUNK = E_PAD // 2048


def _sc_scatter_body(alpha_hbm, m_hbm, src_hbm, dst_hbm, h_hbm,
                     numer_hbm, denp_hbm,
                     acc, accd, src_v, dst_v, alpha_v, sbuf, dbuf, ebuf,
                     rows_v, m_v, gsem, *, f):
    cid = lax.axis_index("c")
    sid = lax.axis_index("s")
    wid = sid * 2 + cid
    sb = wid * STRIPE
    nq = f // 16
    zf = jnp.zeros((16,), jnp.float32)
    zi = jnp.zeros((16,), jnp.int32)

    def zacc(r, _):
        acc[pl.ds(r * 16, 16)] = zf
        return 0

    lax.fori_loop(0, STRIPE * f // 16, zacc, 0)

    def zaccd(r, _):
        accd[pl.ds(r * 16, 16)] = zf
        return 0

    lax.fori_loop(0, STRIPE, zaccd, 0)

    def zbuf(r, _):
        sbuf[pl.ds(r * 16, 16)] = zi
        dbuf[pl.ds(r * 16, 16)] = zi
        ebuf[pl.ds(r * 16, 16)] = zf
        return 0

    lax.fori_loop(0, 128, zbuf, 0)
    pltpu.sync_copy(m_hbm, m_v)
    mvec = m_v[...]
    iota16 = lax.broadcasted_iota(jnp.int32, (16,), 0)

    def chunk(k, _):
        roff = k * CHR
        pltpu.sync_copy(src_hbm.at[pl.ds(roff, CHR)], src_v)
        pltpu.sync_copy(dst_hbm.at[pl.ds(roff, CHR)], dst_v)
        pltpu.sync_copy(alpha_hbm.at[pl.ds(roff, CHR)], alpha_v)

        def vec(j, off_v):
            s = src_v[j // 8, pl.ds((j % 8) * 16, 16)]
            d = dst_v[j // 8, pl.ds((j % 8) * 16, 16)]
            a = alpha_v[j // 8, pl.ds((j % 8) * 16, 16)]
            e = jnp.exp(a - mvec)
            dl = d - sb
            mask = (dl >= 0) & (dl < STRIPE)
            cs = plsc.cumsum(mask.astype(jnp.int32))
            p = off_v + cs - 1
            plsc.store_scatter(sbuf, [p], s, mask=mask)
            plsc.store_scatter(dbuf, [p], dl, mask=mask)
            plsc.store_scatter(ebuf, [p], e, mask=mask)
            return off_v + plsc.all_reduce_population_count(mask)

        off_v = lax.fori_loop(0, CHR * 8, vec,
                              jnp.zeros((16,), jnp.int32), unroll=4)
        off = jnp.max(off_v)
        ng = (off + 15) // 16
        nb = (ng + 3) // 4

        def block(b, _):
            gcnt = jnp.minimum(ng - b * 4, 4)

            def issue(g, _):
                sl = pl.ds((b * 4 + g) * 16, 16)
                rl = pl.ds(g * 16, 16)
                pltpu.async_copy(h_hbm.at[sbuf.at[sl]], rows_v.at[rl],
                                 gsem)
                return 0

            lax.fori_loop(0, gcnt, issue, 0)

            def drain(g, _):
                gg = b * 4 + g
                sl = pl.ds(gg * 16, 16)
                rl = pl.ds(g * 16, 16)
                pltpu.make_async_copy(h_hbm.at[sbuf.at[sl]],
                                      rows_v.at[rl], gsem).wait()
                emv = ebuf[sl]
                emv = jnp.where(gg * 16 + iota16 < off, emv, 0.0)
                ebuf[sl] = emv

                def row(r, _):
                    gi = jnp.full((16,), gg * 16 + r, jnp.int32)
                    em_b = plsc.load_gather(ebuf, [gi])
                    dlv = plsc.load_gather(dbuf, [gi])
                    base = dlv * f + iota16
                    for q in range(nq):
                        plsc.addupdate_scatter(
                            acc, [base + q * 16],
                            rows_v[g * 16 + r, pl.ds(q * 16, 16)] * em_b)
                    plsc.addupdate_scatter(accd, [dlv * 16 + iota16],
                                           em_b)
                    return 0

                lax.fori_loop(0, 16, row, 0)
                return 0

            lax.fori_loop(0, gcnt, drain, 0)
            return 0

        lax.fori_loop(0, nb, block, 0)
        return 0

    lax.fori_loop(0, NCHUNK, chunk, 0)
    pltpu.sync_copy(acc, numer_hbm.at[pl.ds(sb * f, STRIPE * f)])
    pltpu.sync_copy(accd, denp_hbm.at[pl.ds(sb * 16, STRIPE * 16)])


def _sc_scatter(alpha2d, m, src2d, dst2d, h, f):
    fn = pl.kernel(
        functools.partial(_sc_scatter_body, f=f),
        out_type=(jax.ShapeDtypeStruct((N_PAD * f,), jnp.float32),
                  jax.ShapeDtypeStruct((N_PAD * 16,), jnp.float32)),
        mesh=_sc_mesh,
        compiler_params=_sc_params,
        scratch_types=[
            pltpu.VMEM((STRIPE * f,), jnp.float32),
            pltpu.VMEM((STRIPE * 16,), jnp.float32),
            pltpu.VMEM((CHR, 128), jnp.int32),
            pltpu.VMEM((CHR, 128), jnp.int32),
            pltpu.VMEM((CHR, 128), jnp.float32),
            pltpu.VMEM((CHR * 128,), jnp.int32),
            pltpu.VMEM((CHR * 128,), jnp.int32),
            pltpu.VMEM((CHR * 128,), jnp.float32),
            pltpu.VMEM((64, f), jnp.float32),
            pltpu.VMEM((16,), jnp.float32),
            pltpu.SemaphoreType.DMA,
        ],
    )
    return fn(alpha2d, m, src2d, dst2d, h)


def kernel(x, edge_index, edge_attr, batch, emb,
           W1, a_s1, a_d1, We1, ae1, b1,
           W2, a_s2, a_d2, We2, ae2, b2,
           W3, a_s3, a_d3, We3, ae3, b3):
    src = jnp.concatenate([edge_index[0],
                           jnp.zeros((E_PAD - E,), jnp.int32)])
    dst = jnp.concatenate([edge_index[1],
                           jnp.full((E_PAD - E,), N + 100, jnp.int32)])

    v8 = jnp.zeros((DE, 8), jnp.float32)
    v8 = v8.at[:, 0].set(We1 @ ae1).at[:, 1].set(We2 @ ae2)
    v8 = v8.at[:, 2].set(We3 @ ae3)
    ea2 = edge_attr.reshape(E // 8, 128)
    bd = jnp.kron(jnp.eye(8, dtype=jnp.float32), v8)
    t16 = jnp.kron(jnp.ones((8, 1), jnp.float32),
                   jnp.eye(16, dtype=jnp.float32))
    c64, clv = _edge_feat(ea2, bd, t16, v8)
    c8 = c64.reshape(E_PAD, 8)

    W3p = jnp.concatenate([W3, jnp.zeros((H, F3 - C), jnp.float32)], 1)
    b3p = jnp.concatenate([b3, jnp.zeros((F3 - C,), jnp.float32)])
    zc = jnp.zeros((F3 - C,), jnp.float32)
    a_s3p = jnp.concatenate([a_s3, zc])
    a_d3p = jnp.concatenate([a_d3, zc])

    h1, pack = _embed_l1(x, emb, W1, a_s1, a_d1, clv)

    src2d = src.reshape(E_PAD // 128, 128)
    dst2d = dst.reshape(E_PAD // 128, 128)

    layer_w = [(W2, a_s2, a_d2, b1, 1, H), (W3p, a_s3p, a_d3p, b2, 2, F3)]
    h, f = h1, H
    for i in range(3):
        c_l = c8[:, i].reshape(E_PAD // 128, 128)
        asc, adc, al = pack[:, 0], pack[:, 1], pack[:, 2:3]
        alpha2d, tmax = _sc_alpha(asc, adc, src2d, dst2d, c_l)
        m = _gmax(tmax, al)
        m16 = m.reshape(16)
        numer, denp = _sc_scatter(alpha2d, m16, src2d, dst2d, h, f)
        numer = numer.reshape(N_PAD, f)
        denp = denp.reshape(N_PAD, 16)
        if i < 2:
            w_n, as_n, ad_n, b_l, li, f_n = layer_w[i]
            h, pack = _post_layer(numer, denp, al, m, h, b_l,
                                  w_n, as_n, ad_n, clv, li, f_n)
            f = f_n
        else:
            bat_row = batch[None, :]
            out = _final(numer, denp, al, m, h, b3p, bat_row)
    return out


# speedup vs baseline: 7.2645x; 1.1611x over previous
import functools
import jax
import jax.numpy as jnp
from jax import lax
from jax.experimental import pallas as pl
from jax.experimental.pallas import tpu as pltpu
from jax.experimental.pallas import tpu_sc as plsc

N = 10000
E = 320000
F_IN = 128
EMB = 50
H = 256
C = 40
DE = 16
G = 64

N_PAD = 10240
E_PAD = 327680
N_HALF = 5120
SPLIT = 5000
F3 = 128
NEG = -1e30


def _edge_feat_body(ea2_ref, bd_ref, t16_ref, v8_ref, c_ref, clv_ref):
    ea2 = ea2_ref[...]
    pad = jnp.zeros((E_PAD // 8 - E // 8, 128), jnp.float32)
    c_ref[...] = jnp.concatenate([ea2, pad], axis=0) @ bd_ref[...]
    s128 = jnp.sum(ea2, axis=0, keepdims=True)
    clv_ref[...] = ((s128 @ t16_ref[...]) / E) @ v8_ref[...]


def _edge_feat(ea2, bd, t16, v8):
    return pl.pallas_call(
        _edge_feat_body,
        out_shape=(
            jax.ShapeDtypeStruct((E_PAD // 8, 64), jnp.float32),
            jax.ShapeDtypeStruct((1, 8), jnp.float32),
        ),
    )(ea2, bd, t16, v8)


def _attn_cols(h, a_s, a_d, clv, li):
    asc = h @ a_s[:, None]
    adc = h @ a_d[:, None]
    al = asc + adc + clv[0, li]
    al = jnp.where(al >= 0, al, 0.2 * al)
    top = jnp.concatenate([asc, adc, al], 1)
    z = jnp.zeros((N_PAD - N, 2), jnp.float32)
    negs = jnp.full((N_PAD - N, 1), NEG, jnp.float32)
    return jnp.concatenate([top, jnp.concatenate([z, negs], 1)], 0)


def _embed_body(x_ref, emb_ref, w_ref, as_ref, ad_ref, clv_ref,
                h_ref, pack_ref):
    x = x_ref[...]
    rm = jnp.max(x, axis=1, keepdims=True)
    ii = lax.broadcasted_iota(jnp.int32, x.shape, 1)
    cand = jnp.where(x == rm, ii, F_IN)
    idx = jnp.min(cand, axis=1, keepdims=True)
    oh = (lax.broadcasted_iota(jnp.int32, (N, F_IN), 1) == idx
          ).astype(jnp.float32)
    h0 = oh @ emb_ref[...]
    h = h0 @ w_ref[...]
    h_ref[...] = h
    pack_ref[...] = _attn_cols(h, as_ref[...], ad_ref[...], clv_ref[...], 0)


def _embed_l1(x, emb, W1, a_s1, a_d1, clv):
    return pl.pallas_call(
        _embed_body,
        out_shape=(
            jax.ShapeDtypeStruct((N, H), jnp.float32),
            jax.ShapeDtypeStruct((N_PAD, 3), jnp.float32),
        ),
    )(x, emb, W1, a_s1, a_d1, clv)


def _gmax_body(tmax_ref, al_ref, m_ref):
    m = jnp.maximum(jnp.max(tmax_ref[...]), jnp.max(al_ref[...]))
    m_ref[...] = jnp.full((1, 16), m, jnp.float32)


def _gmax(tmax, al_loop):
    return pl.pallas_call(
        _gmax_body,
        out_shape=jax.ShapeDtypeStruct((1, 16), jnp.float32),
    )(tmax, al_loop)


def _combine(numer_ref, denp_ref, al_ref, m_ref, h_ref, b_ref):
    m = m_ref[0, 0]
    exl = jnp.exp(al_ref[...][:N] - m)
    den = denp_ref[:N, 0:1]
    nm = numer_ref[:N]
    h = h_ref[...]
    return (nm + exl * h) / (den + exl) + b_ref[...][None, :]


def _post_body(numer_ref, denp_ref, al_ref, m_ref, h_ref, b_ref,
               w_ref, as_ref, ad_ref, clv_ref,
               h2_ref, pack_ref, *, li):
    out = _combine(numer_ref, denp_ref, al_ref, m_ref, h_ref, b_ref)
    out = jnp.maximum(out, 0.0)
    h2 = out @ w_ref[...]
    h2_ref[...] = h2
    pack_ref[...] = _attn_cols(h2, as_ref[...], ad_ref[...], clv_ref[...], li)


def _post_layer(numer, denp, al_loop, m, h, b, w_next, as_next, ad_next,
                clv, li, f_next):
    return pl.pallas_call(
        functools.partial(_post_body, li=li),
        out_shape=(
            jax.ShapeDtypeStruct((N, f_next), jnp.float32),
            jax.ShapeDtypeStruct((N_PAD, 3), jnp.float32),
        ),
    )(numer, denp, al_loop, m, h, b, w_next, as_next, ad_next, clv)


def _final_body(numer_ref, denp_ref, al_ref, m_ref, h_ref, b_ref,
                bat_ref, out_ref):
    out3 = _combine(numer_ref, denp_ref, al_ref, m_ref, h_ref, b_ref)[:, :C]
    oh = (bat_ref[...] == lax.broadcasted_iota(jnp.int32, (G, N), 0)
          ).astype(jnp.float32)
    sums = oh @ out3
    cnt = oh @ jnp.ones((N, 1), jnp.float32)
    pooled = sums / jnp.maximum(cnt, 1.0)
    pm = jnp.max(pooled, axis=1, keepdims=True)
    ex = jnp.exp(pooled - pm)
    out_ref[...] = ex / jnp.sum(ex, axis=1, keepdims=True)


def _final(numer, denp, al_loop, m, h, b3p, bat_row):
    return pl.pallas_call(
        _final_body,
        out_shape=jax.ShapeDtypeStruct((G, C), jnp.float32),
    )(numer, denp, al_loop, m, h, b3p, bat_row)


NW = 32
EPT = E_PAD // NW
ROWS_PT = E_PAD // 128 // NW
CHR = 16
RG = 64

_sc_mesh = plsc.VectorSubcoreMesh(core_axis_name="c", subcore_axis_name="s")
_sc_params = pltpu.CompilerParams(needs_layout_passes=False)


def _sc_alpha_body(asc_hbm, adc_hbm, src_hbm, dst_hbm, c_hbm,
                   alpha_hbm, tmax_hbm,
                   as_v, ad_v, src_v, dst_v, c_v, alpha_v, tm_v):
    cid = lax.axis_index("c")
    sid = lax.axis_index("s")
    wid = sid * 2 + cid
    rbase = wid * ROWS_PT
    pltpu.sync_copy(asc_hbm, as_v)
    pltpu.sync_copy(adc_hbm, ad_v)
    tm_v[...] = jnp.full((16,), NEG, jnp.float32)

    def chunk(k, _):
        roff = rbase + k * CHR
        pltpu.sync_copy(src_hbm.at[pl.ds(roff, CHR)], src_v)
        pltpu.sync_copy(dst_hbm.at[pl.ds(roff, CHR)], dst_v)
        pltpu.sync_copy(c_hbm.at[pl.ds(roff, CHR)], c_v)

        def vec(j, _):
            r = j // 8
            q = (j % 8) * 16
            s = src_v[r, pl.ds(q, 16)]
            d = dst_v[r, pl.ds(q, 16)]
            a = (plsc.load_gather(as_v, [s]) + plsc.load_gather(ad_v, [d])
                 + c_v[r, pl.ds(q, 16)])
            a = jnp.where(a >= 0, a, 0.2 * a)
            alpha_v[r, pl.ds(q, 16)] = a
            tm_v[...] = jnp.maximum(tm_v[...], a)
            return 0

        lax.fori_loop(0, CHR * 8, vec, 0)
        pltpu.sync_copy(alpha_v, alpha_hbm.at[pl.ds(roff, CHR)])
        return 0

    lax.fori_loop(0, ROWS_PT // CHR, chunk, 0)
    pltpu.sync_copy(tm_v, tmax_hbm.at[wid])


def _sc_alpha(asc, adc, src2d, dst2d, c_l):
    fn = pl.kernel(
        _sc_alpha_body,
        out_type=(jax.ShapeDtypeStruct((E_PAD // 128, 128), jnp.float32),
                  jax.ShapeDtypeStruct((NW, 16), jnp.float32)),
        mesh=_sc_mesh,
        compiler_params=_sc_params,
        scratch_types=[
            pltpu.VMEM((N_PAD,), jnp.float32),
            pltpu.VMEM((N_PAD,), jnp.float32),
            pltpu.VMEM((CHR, 128), jnp.int32),
            pltpu.VMEM((CHR, 128), jnp.int32),
            pltpu.VMEM((CHR, 128), jnp.float32),
            pltpu.VMEM((CHR, 128), jnp.float32),
            pltpu.VMEM((16,), jnp.float32),
        ],
    )
    return fn(asc, adc, src2d, dst2d, c_l)


STRIPE = N_PAD // NW
NCHUNK = E_PAD // 2048


def _sc_scatter_body(alpha_hbm, m_hbm, sd_hbm, h_hbm,
                     numer_hbm, denp_hbm,
                     acc, accd, rec_a, rec_b, al_a, al_b,
                     sbuf, dbuf, ebuf, rows_v, m_v, gsem, lsem, *, f):
    cid = lax.axis_index("c")
    sid = lax.axis_index("s")
    wid = sid * 2 + cid
    sb = wid * STRIPE
    nq = f // 16
    zf = jnp.zeros((16,), jnp.float32)
    zi = jnp.zeros((16,), jnp.int32)

    def zacc(r, _):
        acc[pl.ds(r * 16, 16)] = zf
        return 0

    lax.fori_loop(0, STRIPE * f // 16, zacc, 0)

    def zaccd(r, _):
        accd[pl.ds(r * 16, 16)] = zf
        return 0

    lax.fori_loop(0, STRIPE, zaccd, 0)

    def zbuf(r, _):
        sbuf[pl.ds(r * 16, 16)] = zi
        dbuf[pl.ds(r * 16, 16)] = zi
        ebuf[pl.ds(r * 16, 16)] = zf
        return 0

    lax.fori_loop(0, 128, zbuf, 0)
    pltpu.sync_copy(m_hbm, m_v)
    mvec = m_v[...]
    iota16 = lax.broadcasted_iota(jnp.int32, (16,), 0)

    def issue_load(k, rec_v, al_v):
        pltpu.async_copy(sd_hbm.at[pl.ds(k * CHR, CHR)], rec_v, lsem)
        pltpu.async_copy(alpha_hbm.at[pl.ds(k * CHR, CHR)], al_v, lsem)

    def wait_load(k, rec_v, al_v):
        pltpu.make_async_copy(sd_hbm.at[pl.ds(k * CHR, CHR)], rec_v,
                              lsem).wait()
        pltpu.make_async_copy(alpha_hbm.at[pl.ds(k * CHR, CHR)], al_v,
                              lsem).wait()

    def process(k, rec_v, al_v):
        def vec(j, off_v):
            s = rec_v[j // 8, 0, pl.ds((j % 8) * 16, 16)]
            d = rec_v[j // 8, 1, pl.ds((j % 8) * 16, 16)]
            a = al_v[j // 8, pl.ds((j % 8) * 16, 16)]
            e = jnp.exp(a - mvec)
            dl = d - sb
            mask = dl.astype(jnp.uint32) < jnp.uint32(STRIPE)
            cs = plsc.cumsum(mask.astype(jnp.int32))
            p = off_v + cs - 1
            plsc.store_scatter(sbuf, [p], s, mask=mask)
            plsc.store_scatter(dbuf, [p], dl, mask=mask)
            plsc.store_scatter(ebuf, [p], e, mask=mask)
            return off_v + plsc.all_reduce_population_count(mask)

        off_v = lax.fori_loop(0, CHR * 8, vec,
                              jnp.zeros((16,), jnp.int32), unroll=4)
        off = jnp.max(off_v)
        ng = (off + 15) // 16
        nb = (ng + 3) // 4

        def block(b, _):
            gcnt = jnp.minimum(ng - b * 4, 4)

            def issue(g, _):
                sl = pl.ds((b * 4 + g) * 16, 16)
                rl = pl.ds(g * 16, 16)
                pltpu.async_copy(h_hbm.at[sbuf.at[sl]], rows_v.at[rl],
                                 gsem)
                return 0

            lax.fori_loop(0, gcnt, issue, 0)

            def drain(g, _):
                gg = b * 4 + g
                sl = pl.ds(gg * 16, 16)
                rl = pl.ds(g * 16, 16)
                pltpu.make_async_copy(h_hbm.at[sbuf.at[sl]],
                                      rows_v.at[rl], gsem).wait()
                emv = ebuf[sl]
                emv = jnp.where(gg * 16 + iota16 < off, emv, 0.0)
                ebuf[sl] = emv

                def row(r, _):
                    gi = jnp.full((16,), gg * 16 + r, jnp.int32)
                    em_b = plsc.load_gather(ebuf, [gi])
                    dlv = plsc.load_gather(dbuf, [gi])
                    base = dlv * f + iota16
                    for q in range(nq):
                        plsc.addupdate_scatter(
                            acc, [base + q * 16],
                            rows_v[g * 16 + r, pl.ds(q * 16, 16)] * em_b)
                    plsc.addupdate_scatter(accd, [dlv * 16 + iota16],
                                           em_b)
                    return 0

                lax.fori_loop(0, 16, row, 0)
                return 0

            lax.fori_loop(0, gcnt, drain, 0)
            return 0

        lax.fori_loop(0, nb, block, 0)

    issue_load(0, rec_a, al_a)

    def body2(k2, _):
        k = 2 * k2
        issue_load(k + 1, rec_b, al_b)
        wait_load(k, rec_a, al_a)
        process(k, rec_a, al_a)

        @pl.when(k2 < NCHUNK // 2 - 1)
        def _():
            issue_load(k + 2, rec_a, al_a)

        wait_load(k + 1, rec_b, al_b)
        process(k + 1, rec_b, al_b)
        return 0

    lax.fori_loop(0, NCHUNK // 2, body2, 0)
    pltpu.sync_copy(acc, numer_hbm.at[pl.ds(sb * f, STRIPE * f)])
    pltpu.sync_copy(accd, denp_hbm.at[pl.ds(sb * 16, STRIPE * 16)])


def _sc_scatter(alpha2d, m, sd, h, f):
    fn = pl.kernel(
        functools.partial(_sc_scatter_body, f=f),
        out_type=(jax.ShapeDtypeStruct((N_PAD * f,), jnp.float32),
                  jax.ShapeDtypeStruct((N_PAD * 16,), jnp.float32)),
        mesh=_sc_mesh,
        compiler_params=_sc_params,
        scratch_types=[
            pltpu.VMEM((STRIPE * f,), jnp.float32),
            pltpu.VMEM((STRIPE * 16,), jnp.float32),
            pltpu.VMEM((CHR, 2, 128), jnp.int32),
            pltpu.VMEM((CHR, 2, 128), jnp.int32),
            pltpu.VMEM((CHR, 128), jnp.float32),
            pltpu.VMEM((CHR, 128), jnp.float32),
            pltpu.VMEM((CHR * 128,), jnp.int32),
            pltpu.VMEM((CHR * 128,), jnp.int32),
            pltpu.VMEM((CHR * 128,), jnp.float32),
            pltpu.VMEM((64, f), jnp.float32),
            pltpu.VMEM((16,), jnp.float32),
            pltpu.SemaphoreType.DMA,
            pltpu.SemaphoreType.DMA,
        ],
    )
    return fn(alpha2d, m, sd, h)


def kernel(x, edge_index, edge_attr, batch, emb,
           W1, a_s1, a_d1, We1, ae1, b1,
           W2, a_s2, a_d2, We2, ae2, b2,
           W3, a_s3, a_d3, We3, ae3, b3):
    src = jnp.concatenate([edge_index[0],
                           jnp.zeros((E_PAD - E,), jnp.int32)])
    dst = jnp.concatenate([edge_index[1],
                           jnp.full((E_PAD - E,), N + 100, jnp.int32)])

    v8 = jnp.zeros((DE, 8), jnp.float32)
    v8 = v8.at[:, 0].set(We1 @ ae1).at[:, 1].set(We2 @ ae2)
    v8 = v8.at[:, 2].set(We3 @ ae3)
    ea2 = edge_attr.reshape(E // 8, 128)
    bd = jnp.kron(jnp.eye(8, dtype=jnp.float32), v8)
    t16 = jnp.kron(jnp.ones((8, 1), jnp.float32),
                   jnp.eye(16, dtype=jnp.float32))
    c64, clv = _edge_feat(ea2, bd, t16, v8)
    c8 = c64.reshape(E_PAD, 8)

    W3p = jnp.concatenate([W3, jnp.zeros((H, F3 - C), jnp.float32)], 1)
    b3p = jnp.concatenate([b3, jnp.zeros((F3 - C,), jnp.float32)])
    zc = jnp.zeros((F3 - C,), jnp.float32)
    a_s3p = jnp.concatenate([a_s3, zc])
    a_d3p = jnp.concatenate([a_d3, zc])

    h1, pack = _embed_l1(x, emb, W1, a_s1, a_d1, clv)

    src2d = src.reshape(E_PAD // 128, 128)
    dst2d = dst.reshape(E_PAD // 128, 128)
    sd = jnp.stack([src2d, dst2d], axis=1)

    layer_w = [(W2, a_s2, a_d2, b1, 1, H), (W3p, a_s3p, a_d3p, b2, 2, F3)]
    h, f = h1, H
    for i in range(3):
        c_l = c8[:, i].reshape(E_PAD // 128, 128)
        asc, adc, al = pack[:, 0], pack[:, 1], pack[:, 2:3]
        alpha2d, tmax = _sc_alpha(asc, adc, src2d, dst2d, c_l)
        m = _gmax(tmax, al)
        m16 = m.reshape(16)
        numer, denp = _sc_scatter(alpha2d, m16, sd, h, f)
        numer = numer.reshape(N_PAD, f)
        denp = denp.reshape(N_PAD, 16)
        if i < 2:
            w_n, as_n, ad_n, b_l, li, f_n = layer_w[i]
            h, pack = _post_layer(numer, denp, al, m, h, b_l,
                                  w_n, as_n, ad_n, clv, li, f_n)
            f = f_n
        else:
            bat_row = batch[None, :]
            out = _final(numer, denp, al, m, h, b3p, bat_row)
    return out


# speedup vs baseline: 7.8147x; 1.0757x over previous
import functools
import jax
import jax.numpy as jnp
from jax import lax
from jax.experimental import pallas as pl
from jax.experimental.pallas import tpu as pltpu
from jax.experimental.pallas import tpu_sc as plsc

N = 10000
E = 320000
F_IN = 128
EMB = 50
H = 256
C = 40
DE = 16
G = 64

N_PAD = 10240
E_PAD = 327680
N_HALF = 5120
SPLIT = 5000
F3 = 128
NEG = -1e30


def _edge_feat_body(ea2_ref, bd_ref, t16_ref, v8_ref, c_ref, clv_ref):
    ea2 = ea2_ref[...]
    pad = jnp.zeros((E_PAD // 8 - E // 8, 128), jnp.float32)
    c_ref[...] = jnp.concatenate([ea2, pad], axis=0) @ bd_ref[...]
    s128 = jnp.sum(ea2, axis=0, keepdims=True)
    clv_ref[...] = ((s128 @ t16_ref[...]) / E) @ v8_ref[...]


def _edge_feat(ea2, bd, t16, v8):
    return pl.pallas_call(
        _edge_feat_body,
        out_shape=(
            jax.ShapeDtypeStruct((E_PAD // 8, 64), jnp.float32),
            jax.ShapeDtypeStruct((1, 8), jnp.float32),
        ),
    )(ea2, bd, t16, v8)


def _attn_cols(h, a_s, a_d, clv, li):
    asc = h @ a_s[:, None]
    adc = h @ a_d[:, None]
    al = asc + adc + clv[0, li]
    al = jnp.where(al >= 0, al, 0.2 * al)
    top = jnp.concatenate([asc, adc, al], 1)
    z = jnp.zeros((N_PAD - N, 2), jnp.float32)
    negs = jnp.full((N_PAD - N, 1), NEG, jnp.float32)
    return jnp.concatenate([top, jnp.concatenate([z, negs], 1)], 0)


def _embed_body(x_ref, emb_ref, w_ref, as_ref, ad_ref, clv_ref,
                h_ref, pack_ref):
    x = x_ref[...]
    rm = jnp.max(x, axis=1, keepdims=True)
    ii = lax.broadcasted_iota(jnp.int32, x.shape, 1)
    cand = jnp.where(x == rm, ii, F_IN)
    idx = jnp.min(cand, axis=1, keepdims=True)
    oh = (lax.broadcasted_iota(jnp.int32, (N, F_IN), 1) == idx
          ).astype(jnp.float32)
    h0 = oh @ emb_ref[...]
    h = h0 @ w_ref[...]
    h_ref[...] = h
    pack_ref[...] = _attn_cols(h, as_ref[...], ad_ref[...], clv_ref[...], 0)


def _embed_l1(x, emb, W1, a_s1, a_d1, clv):
    return pl.pallas_call(
        _embed_body,
        out_shape=(
            jax.ShapeDtypeStruct((N, H), jnp.float32),
            jax.ShapeDtypeStruct((N_PAD, 3), jnp.float32),
        ),
    )(x, emb, W1, a_s1, a_d1, clv)


def _gmax_body(tmax_ref, al_ref, m_ref):
    m = jnp.maximum(jnp.max(tmax_ref[...]), jnp.max(al_ref[...]))
    m_ref[...] = jnp.full((1, 16), m, jnp.float32)


def _gmax(tmax, al_loop):
    return pl.pallas_call(
        _gmax_body,
        out_shape=jax.ShapeDtypeStruct((1, 16), jnp.float32),
    )(tmax, al_loop)


def _combine(numer_ref, denp_ref, al_ref, m_ref, h_ref, b_ref):
    m = m_ref[0, 0]
    exl = jnp.exp(al_ref[...][:N] - m)
    den = denp_ref[:N, 0:1]
    nm = numer_ref[:N]
    h = h_ref[...]
    return (nm + exl * h) / (den + exl) + b_ref[...][None, :]


def _post_body(numer_ref, denp_ref, al_ref, m_ref, h_ref, b_ref,
               w_ref, as_ref, ad_ref, clv_ref,
               h2_ref, pack_ref, *, li):
    out = _combine(numer_ref, denp_ref, al_ref, m_ref, h_ref, b_ref)
    out = jnp.maximum(out, 0.0)
    h2 = out @ w_ref[...]
    h2_ref[...] = h2
    pack_ref[...] = _attn_cols(h2, as_ref[...], ad_ref[...], clv_ref[...], li)


def _post_layer(numer, denp, al_loop, m, h, b, w_next, as_next, ad_next,
                clv, li, f_next):
    return pl.pallas_call(
        functools.partial(_post_body, li=li),
        out_shape=(
            jax.ShapeDtypeStruct((N, f_next), jnp.float32),
            jax.ShapeDtypeStruct((N_PAD, 3), jnp.float32),
        ),
    )(numer, denp, al_loop, m, h, b, w_next, as_next, ad_next, clv)


def _final_body(numer_ref, denp_ref, al_ref, m_ref, h_ref, b_ref,
                bat_ref, out_ref):
    out3 = _combine(numer_ref, denp_ref, al_ref, m_ref, h_ref, b_ref)[:, :C]
    oh = (bat_ref[...] == lax.broadcasted_iota(jnp.int32, (G, N), 0)
          ).astype(jnp.float32)
    sums = oh @ out3
    cnt = oh @ jnp.ones((N, 1), jnp.float32)
    pooled = sums / jnp.maximum(cnt, 1.0)
    pm = jnp.max(pooled, axis=1, keepdims=True)
    ex = jnp.exp(pooled - pm)
    out_ref[...] = ex / jnp.sum(ex, axis=1, keepdims=True)


def _final(numer, denp, al_loop, m, h, b3p, bat_row):
    return pl.pallas_call(
        _final_body,
        out_shape=jax.ShapeDtypeStruct((G, C), jnp.float32),
    )(numer, denp, al_loop, m, h, b3p, bat_row)


NW = 32
EPT = E_PAD // NW
ROWS_PT = E_PAD // 128 // NW
CHR = 16
RG = 64

_sc_mesh = plsc.VectorSubcoreMesh(core_axis_name="c", subcore_axis_name="s")
_sc_params = pltpu.CompilerParams(needs_layout_passes=False)


def _sc_alpha_body(asc_hbm, adc_hbm, src_hbm, dst_hbm, c_hbm,
                   alpha_hbm, tmax_hbm,
                   as_v, ad_v, src_v, dst_v, c_v, alpha_v, tm_v):
    cid = lax.axis_index("c")
    sid = lax.axis_index("s")
    wid = sid * 2 + cid
    rbase = wid * ROWS_PT
    pltpu.sync_copy(asc_hbm, as_v)
    pltpu.sync_copy(adc_hbm, ad_v)
    tm_v[...] = jnp.full((16,), NEG, jnp.float32)

    def chunk(k, _):
        roff = rbase + k * CHR
        pltpu.sync_copy(src_hbm.at[pl.ds(roff, CHR)], src_v)
        pltpu.sync_copy(dst_hbm.at[pl.ds(roff, CHR)], dst_v)
        pltpu.sync_copy(c_hbm.at[pl.ds(roff, CHR)], c_v)

        def vec(j, _):
            r = j // 8
            q = (j % 8) * 16
            s = src_v[r, pl.ds(q, 16)]
            d = jnp.minimum(dst_v[r, pl.ds(q, 16)], N_PAD - 1)
            a = (plsc.load_gather(as_v, [s]) + plsc.load_gather(ad_v, [d])
                 + c_v[r, pl.ds(q, 16)])
            a = jnp.where(a >= 0, a, 0.2 * a)
            alpha_v[r, pl.ds(q, 16)] = a
            tm_v[...] = jnp.maximum(tm_v[...], a)
            return 0

        lax.fori_loop(0, CHR * 8, vec, 0)
        pltpu.sync_copy(alpha_v, alpha_hbm.at[pl.ds(roff, CHR)])
        return 0

    lax.fori_loop(0, ROWS_PT // CHR, chunk, 0)
    pltpu.sync_copy(tm_v, tmax_hbm.at[wid])


def _sc_alpha(asc, adc, src2d, dst2d, c_l):
    fn = pl.kernel(
        _sc_alpha_body,
        out_type=(jax.ShapeDtypeStruct((E_PAD // 128, 128), jnp.float32),
                  jax.ShapeDtypeStruct((NW, 16), jnp.float32)),
        mesh=_sc_mesh,
        compiler_params=_sc_params,
        scratch_types=[
            pltpu.VMEM((N_PAD,), jnp.float32),
            pltpu.VMEM((N_PAD,), jnp.float32),
            pltpu.VMEM((CHR, 128), jnp.int32),
            pltpu.VMEM((CHR, 128), jnp.int32),
            pltpu.VMEM((CHR, 128), jnp.float32),
            pltpu.VMEM((CHR, 128), jnp.float32),
            pltpu.VMEM((16,), jnp.float32),
        ],
    )
    return fn(asc, adc, src2d, dst2d, c_l)


STRIPE = N_PAD // NW
NCHUNK = E_PAD // 2048


def _sc_scatter_body(alpha_hbm, m_hbm, sd_hbm, h_hbm,
                     numer_hbm, denp_hbm,
                     acc, accd, rec_a, rec_b, al_a, al_b,
                     sbuf, dbuf, ebuf, rows_v, m_v, gsem, lsem, *, f):
    cid = lax.axis_index("c")
    sid = lax.axis_index("s")
    wid = sid * 2 + cid
    sb = wid * STRIPE
    nq = f // 16
    zf = jnp.zeros((16,), jnp.float32)
    zi = jnp.zeros((16,), jnp.int32)
    iota16 = lax.broadcasted_iota(jnp.int32, (16,), 0)

    def zacc(r, _):
        acc[pl.ds(r * 16, 16)] = zf
        return 0

    lax.fori_loop(0, STRIPE * f // 16, zacc, 0)

    def zaccd(r, _):
        accd[pl.ds(r * 16, 16)] = zf
        return 0

    lax.fori_loop(0, STRIPE, zaccd, 0)

    def zbuf(r, _):
        sbuf[pl.ds(r * 16, 16)] = r * 16 + iota16
        dbuf[pl.ds(r * 16, 16)] = zi
        ebuf[pl.ds(r * 16, 16)] = zf
        return 0

    lax.fori_loop(0, 128, zbuf, 0)
    pltpu.sync_copy(m_hbm, m_v)
    mvec = m_v[...]

    def issue_load(k, rec_v, al_v):
        pltpu.async_copy(sd_hbm.at[pl.ds(k * CHR, CHR)], rec_v, lsem)
        pltpu.async_copy(alpha_hbm.at[pl.ds(k * CHR, CHR)], al_v, lsem)

    def wait_load(k, rec_v, al_v):
        pltpu.make_async_copy(sd_hbm.at[pl.ds(k * CHR, CHR)], rec_v,
                              lsem).wait()
        pltpu.make_async_copy(alpha_hbm.at[pl.ds(k * CHR, CHR)], al_v,
                              lsem).wait()

    def process(k, rec_v, al_v):
        def vec(j, off_v):
            s = rec_v[j // 8, 0, pl.ds((j % 8) * 16, 16)]
            d = rec_v[j // 8, 1, pl.ds((j % 8) * 16, 16)]
            a = al_v[j // 8, pl.ds((j % 8) * 16, 16)]
            e = jnp.exp(a - mvec)
            dl = d - sb
            mask = dl.astype(jnp.uint32) < jnp.uint32(STRIPE)
            cs = plsc.cumsum(mask.astype(jnp.int32))
            p = off_v + cs - 1
            plsc.store_scatter(sbuf, [p], s, mask=mask)
            plsc.store_scatter(dbuf, [p], dl, mask=mask)
            plsc.store_scatter(ebuf, [p], e, mask=mask)
            return off_v + plsc.all_reduce_population_count(mask)

        off_v = lax.fori_loop(0, CHR * 8, vec,
                              jnp.zeros((16,), jnp.int32), unroll=4)
        off = jnp.max(off_v)
        ng = (off + 15) // 16
        nb = (ng + 3) // 4

        def block(b, _):
            gcnt = jnp.minimum(ng - b * 4, 4)

            def issue(g, _):
                sl = pl.ds((b * 4 + g) * 16, 16)
                rl = pl.ds(g * 16, 16)
                pltpu.async_copy(h_hbm.at[sbuf.at[sl]], rows_v.at[rl],
                                 gsem)
                return 0

            lax.fori_loop(0, gcnt, issue, 0)

            def drain(g, _):
                gg = b * 4 + g
                sl = pl.ds(gg * 16, 16)
                rl = pl.ds(g * 16, 16)
                pltpu.make_async_copy(h_hbm.at[sbuf.at[sl]],
                                      rows_v.at[rl], gsem).wait()
                emv = ebuf[sl]
                emv = jnp.where(gg * 16 + iota16 < off, emv, 0.0)
                ebuf[sl] = emv

                def row(r, _):
                    gi = jnp.full((16,), gg * 16 + r, jnp.int32)
                    em_b = plsc.load_gather(ebuf, [gi])
                    dlv = plsc.load_gather(dbuf, [gi])
                    base = dlv * f + iota16
                    for q in range(nq):
                        plsc.addupdate_scatter(
                            acc, [base + q * 16],
                            rows_v[g * 16 + r, pl.ds(q * 16, 16)] * em_b)
                    plsc.addupdate_scatter(accd, [dlv * 16 + iota16],
                                           em_b)
                    return 0

                lax.fori_loop(0, 16, row, 0, unroll=4)
                return 0

            lax.fori_loop(0, gcnt, drain, 0)
            return 0

        lax.fori_loop(0, nb, block, 0)

    issue_load(0, rec_a, al_a)

    def body2(k2, _):
        k = 2 * k2
        issue_load(k + 1, rec_b, al_b)
        wait_load(k, rec_a, al_a)
        process(k, rec_a, al_a)

        @pl.when(k2 < NCHUNK // 2 - 1)
        def _():
            issue_load(k + 2, rec_a, al_a)

        wait_load(k + 1, rec_b, al_b)
        process(k + 1, rec_b, al_b)
        return 0

    lax.fori_loop(0, NCHUNK // 2, body2, 0)
    pltpu.sync_copy(acc, numer_hbm.at[pl.ds(sb * f, STRIPE * f)])
    pltpu.sync_copy(accd, denp_hbm.at[pl.ds(sb * 16, STRIPE * 16)])


def _sc_scatter(alpha2d, m, sd, h, f):
    fn = pl.kernel(
        functools.partial(_sc_scatter_body, f=f),
        out_type=(jax.ShapeDtypeStruct((N_PAD * f,), jnp.float32),
                  jax.ShapeDtypeStruct((N_PAD * 16,), jnp.float32)),
        mesh=_sc_mesh,
        compiler_params=_sc_params,
        scratch_types=[
            pltpu.VMEM((STRIPE * f,), jnp.float32),
            pltpu.VMEM((STRIPE * 16,), jnp.float32),
            pltpu.VMEM((CHR, 2, 128), jnp.int32),
            pltpu.VMEM((CHR, 2, 128), jnp.int32),
            pltpu.VMEM((CHR, 128), jnp.float32),
            pltpu.VMEM((CHR, 128), jnp.float32),
            pltpu.VMEM((CHR * 128,), jnp.int32),
            pltpu.VMEM((CHR * 128,), jnp.int32),
            pltpu.VMEM((CHR * 128,), jnp.float32),
            pltpu.VMEM((64, f), jnp.float32),
            pltpu.VMEM((16,), jnp.float32),
            pltpu.SemaphoreType.DMA,
            pltpu.SemaphoreType.DMA,
        ],
    )
    return fn(alpha2d, m, sd, h)


def kernel(x, edge_index, edge_attr, batch, emb,
           W1, a_s1, a_d1, We1, ae1, b1,
           W2, a_s2, a_d2, We2, ae2, b2,
           W3, a_s3, a_d3, We3, ae3, b3):
    src = jnp.concatenate([edge_index[0],
                           jnp.zeros((E_PAD - E,), jnp.int32)])
    dst = jnp.concatenate([edge_index[1],
                           jnp.full((E_PAD - E,), 1 << 30, jnp.int32)])

    v8 = jnp.zeros((DE, 8), jnp.float32)
    v8 = v8.at[:, 0].set(We1 @ ae1).at[:, 1].set(We2 @ ae2)
    v8 = v8.at[:, 2].set(We3 @ ae3)
    ea2 = edge_attr.reshape(E // 8, 128)
    bd = jnp.kron(jnp.eye(8, dtype=jnp.float32), v8)
    t16 = jnp.kron(jnp.ones((8, 1), jnp.float32),
                   jnp.eye(16, dtype=jnp.float32))
    c64, clv = _edge_feat(ea2, bd, t16, v8)
    c8 = c64.reshape(E_PAD, 8)

    W3p = jnp.concatenate([W3, jnp.zeros((H, F3 - C), jnp.float32)], 1)
    b3p = jnp.concatenate([b3, jnp.zeros((F3 - C,), jnp.float32)])
    zc = jnp.zeros((F3 - C,), jnp.float32)
    a_s3p = jnp.concatenate([a_s3, zc])
    a_d3p = jnp.concatenate([a_d3, zc])

    h1, pack = _embed_l1(x, emb, W1, a_s1, a_d1, clv)

    src2d = src.reshape(E_PAD // 128, 128)
    dst2d = dst.reshape(E_PAD // 128, 128)
    sd = jnp.stack([src2d, dst2d], axis=1)

    layer_w = [(W2, a_s2, a_d2, b1, 1, H), (W3p, a_s3p, a_d3p, b2, 2, F3)]
    h, f = h1, H
    for i in range(3):
        c_l = c8[:, i].reshape(E_PAD // 128, 128)
        asc, adc, al = pack[:, 0], pack[:, 1], pack[:, 2:3]
        alpha2d, tmax = _sc_alpha(asc, adc, src2d, dst2d, c_l)
        m = _gmax(tmax, al)
        m16 = m.reshape(16)
        numer, denp = _sc_scatter(alpha2d, m16, sd, h, f)
        numer = numer.reshape(N_PAD, f)
        denp = denp.reshape(N_PAD, 16)
        if i < 2:
            w_n, as_n, ad_n, b_l, li, f_n = layer_w[i]
            h, pack = _post_layer(numer, denp, al, m, h, b_l,
                                  w_n, as_n, ad_n, clv, li, f_n)
            f = f_n
        else:
            bat_row = batch[None, :]
            out = _final(numer, denp, al, m, h, b3p, bat_row)
    return out


# speedup vs baseline: 8.0509x; 1.0302x over previous
import functools
import jax
import jax.numpy as jnp
from jax import lax
from jax.experimental import pallas as pl
from jax.experimental.pallas import tpu as pltpu
from jax.experimental.pallas import tpu_sc as plsc

N = 10000
E = 320000
F_IN = 128
EMB = 50
H = 256
C = 40
DE = 16
G = 64

N_PAD = 10240
E_PAD = 327680
N_HALF = 5120
SPLIT = 5000
F3 = 128
NEG = -1e30


def _edge_feat_body(ea2_ref, bd_ref, t16_ref, v8_ref, c_ref, clv_ref):
    ea2 = ea2_ref[...]
    pad = jnp.zeros((E_PAD // 8 - E // 8, 128), jnp.float32)
    c_ref[...] = jnp.concatenate([ea2, pad], axis=0) @ bd_ref[...]
    s128 = jnp.sum(ea2, axis=0, keepdims=True)
    clv_ref[...] = ((s128 @ t16_ref[...]) / E) @ v8_ref[...]


def _edge_feat(ea2, bd, t16, v8):
    return pl.pallas_call(
        _edge_feat_body,
        out_shape=(
            jax.ShapeDtypeStruct((E_PAD // 8, 64), jnp.float32),
            jax.ShapeDtypeStruct((1, 8), jnp.float32),
        ),
    )(ea2, bd, t16, v8)


def _attn_cols(h, a_s, a_d, clv, li):
    asc = h @ a_s[:, None]
    adc = h @ a_d[:, None]
    al = asc + adc + clv[0, li]
    al = jnp.where(al >= 0, al, 0.2 * al)
    top = jnp.concatenate([asc, adc, al], 1)
    z = jnp.zeros((N_PAD - N, 2), jnp.float32)
    negs = jnp.full((N_PAD - N, 1), NEG, jnp.float32)
    return jnp.concatenate([top, jnp.concatenate([z, negs], 1)], 0)


def _embed_body(x_ref, emb_ref, w_ref, as_ref, ad_ref, clv_ref,
                h_ref, pack_ref):
    x = x_ref[...]
    rm = jnp.max(x, axis=1, keepdims=True)
    ii = lax.broadcasted_iota(jnp.int32, x.shape, 1)
    cand = jnp.where(x == rm, ii, F_IN)
    idx = jnp.min(cand, axis=1, keepdims=True)
    oh = (lax.broadcasted_iota(jnp.int32, (N, F_IN), 1) == idx
          ).astype(jnp.float32)
    h0 = oh @ emb_ref[...]
    h = h0 @ w_ref[...]
    h_ref[...] = h
    pack_ref[...] = _attn_cols(h, as_ref[...], ad_ref[...], clv_ref[...], 0)


def _embed_l1(x, emb, W1, a_s1, a_d1, clv):
    return pl.pallas_call(
        _embed_body,
        out_shape=(
            jax.ShapeDtypeStruct((N, H), jnp.float32),
            jax.ShapeDtypeStruct((N_PAD, 3), jnp.float32),
        ),
    )(x, emb, W1, a_s1, a_d1, clv)


def _gmax_body(tmax_ref, al_ref, m_ref):
    m = jnp.maximum(jnp.max(tmax_ref[...]), jnp.max(al_ref[...]))
    m_ref[...] = jnp.full((1, 16), m, jnp.float32)


def _gmax(tmax, al_loop):
    return pl.pallas_call(
        _gmax_body,
        out_shape=jax.ShapeDtypeStruct((1, 16), jnp.float32),
    )(tmax, al_loop)


def _combine(numer_ref, denp_ref, al_ref, m_ref, h_ref, b_ref):
    m = m_ref[0, 0]
    exl = jnp.exp(al_ref[...][:N] - m)
    den = denp_ref[:N, 0:1]
    nm = numer_ref[:N]
    h = h_ref[...]
    return (nm + exl * h) / (den + exl) + b_ref[...][None, :]


def _post_body(numer_ref, denp_ref, al_ref, m_ref, h_ref, b_ref,
               w_ref, as_ref, ad_ref, clv_ref,
               h2_ref, pack_ref, *, li):
    out = _combine(numer_ref, denp_ref, al_ref, m_ref, h_ref, b_ref)
    out = jnp.maximum(out, 0.0)
    h2 = out @ w_ref[...]
    h2_ref[...] = h2
    pack_ref[...] = _attn_cols(h2, as_ref[...], ad_ref[...], clv_ref[...], li)


def _post_layer(numer, denp, al_loop, m, h, b, w_next, as_next, ad_next,
                clv, li, f_next):
    return pl.pallas_call(
        functools.partial(_post_body, li=li),
        out_shape=(
            jax.ShapeDtypeStruct((N, f_next), jnp.float32),
            jax.ShapeDtypeStruct((N_PAD, 3), jnp.float32),
        ),
    )(numer, denp, al_loop, m, h, b, w_next, as_next, ad_next, clv)


def _final_body(numer_ref, denp_ref, al_ref, m_ref, h_ref, b_ref,
                bat_ref, out_ref):
    out3 = _combine(numer_ref, denp_ref, al_ref, m_ref, h_ref, b_ref)[:, :C]
    oh = (bat_ref[...] == lax.broadcasted_iota(jnp.int32, (G, N), 0)
          ).astype(jnp.float32)
    sums = oh @ out3
    cnt = oh @ jnp.ones((N, 1), jnp.float32)
    pooled = sums / jnp.maximum(cnt, 1.0)
    pm = jnp.max(pooled, axis=1, keepdims=True)
    ex = jnp.exp(pooled - pm)
    out_ref[...] = ex / jnp.sum(ex, axis=1, keepdims=True)


def _final(numer, denp, al_loop, m, h, b3p, bat_row):
    return pl.pallas_call(
        _final_body,
        out_shape=jax.ShapeDtypeStruct((G, C), jnp.float32),
    )(numer, denp, al_loop, m, h, b3p, bat_row)


NW = 32
EPT = E_PAD // NW
ROWS_PT = E_PAD // 128 // NW
CHR = 16
RG = 64

_sc_mesh = plsc.VectorSubcoreMesh(core_axis_name="c", subcore_axis_name="s")
_sc_params = pltpu.CompilerParams(needs_layout_passes=False)


def _sc_alpha_body(asc_hbm, adc_hbm, src_hbm, dst_hbm, c_hbm,
                   alpha_hbm, tmax_hbm,
                   as_v, ad_v, src_v, dst_v, c_v, alpha_v, tm_v):
    cid = lax.axis_index("c")
    sid = lax.axis_index("s")
    wid = sid * 2 + cid
    rbase = wid * ROWS_PT
    pltpu.sync_copy(asc_hbm, as_v)
    pltpu.sync_copy(adc_hbm, ad_v)
    tm_v[...] = jnp.full((16,), NEG, jnp.float32)

    def chunk(k, _):
        roff = rbase + k * CHR
        pltpu.sync_copy(src_hbm.at[pl.ds(roff, CHR)], src_v)
        pltpu.sync_copy(dst_hbm.at[pl.ds(roff, CHR)], dst_v)
        pltpu.sync_copy(c_hbm.at[pl.ds(roff, CHR)], c_v)

        def vec(j, _):
            r = j // 8
            q = (j % 8) * 16
            s = src_v[r, pl.ds(q, 16)]
            d = jnp.minimum(dst_v[r, pl.ds(q, 16)], N_PAD - 1)
            a = (plsc.load_gather(as_v, [s]) + plsc.load_gather(ad_v, [d])
                 + c_v[r, pl.ds(q, 16)])
            a = jnp.where(a >= 0, a, 0.2 * a)
            alpha_v[r, pl.ds(q, 16)] = a
            tm_v[...] = jnp.maximum(tm_v[...], a)
            return 0

        lax.fori_loop(0, CHR * 8, vec, 0)
        pltpu.sync_copy(alpha_v, alpha_hbm.at[pl.ds(roff, CHR)])
        return 0

    lax.fori_loop(0, ROWS_PT // CHR, chunk, 0)
    pltpu.sync_copy(tm_v, tmax_hbm.at[wid])


def _sc_alpha(asc, adc, src2d, dst2d, c_l):
    fn = pl.kernel(
        _sc_alpha_body,
        out_type=(jax.ShapeDtypeStruct((E_PAD // 128, 128), jnp.float32),
                  jax.ShapeDtypeStruct((NW, 16), jnp.float32)),
        mesh=_sc_mesh,
        compiler_params=_sc_params,
        scratch_types=[
            pltpu.VMEM((N_PAD,), jnp.float32),
            pltpu.VMEM((N_PAD,), jnp.float32),
            pltpu.VMEM((CHR, 128), jnp.int32),
            pltpu.VMEM((CHR, 128), jnp.int32),
            pltpu.VMEM((CHR, 128), jnp.float32),
            pltpu.VMEM((CHR, 128), jnp.float32),
            pltpu.VMEM((16,), jnp.float32),
        ],
    )
    return fn(asc, adc, src2d, dst2d, c_l)


STRIPE = N_PAD // NW
NCHUNK = E_PAD // 2048


def _sc_scatter_body(alpha_hbm, m_hbm, sd_hbm, h_hbm,
                     numer_hbm, denp_hbm,
                     acc, accd, rec_a, rec_b, al_a, al_b,
                     sbuf, dbuf, ebuf, rows_v, m_v, gsem, lsem, *, f):
    cid = lax.axis_index("c")
    sid = lax.axis_index("s")
    wid = sid * 2 + cid
    sb = wid * STRIPE
    nq = f // 16
    zf = jnp.zeros((16,), jnp.float32)
    zi = jnp.zeros((16,), jnp.int32)
    iota16 = lax.broadcasted_iota(jnp.int32, (16,), 0)

    def zacc(r, _):
        for q in range(nq):
            acc[r, pl.ds(q * 16, 16)] = zf
        accd[pl.ds(r * 16, 16)] = zf
        return 0

    lax.fori_loop(0, STRIPE, zacc, 0)

    def zbuf(r, _):
        sbuf[pl.ds(r * 16, 16)] = r * 16 + iota16
        dbuf[pl.ds(r * 16, 16)] = zi
        ebuf[pl.ds(r * 16, 16)] = zf
        return 0

    lax.fori_loop(0, 128, zbuf, 0)
    pltpu.sync_copy(m_hbm, m_v)
    mvec = m_v[...]

    def issue_load(k, rec_v, al_v):
        pltpu.async_copy(sd_hbm.at[pl.ds(k * CHR, CHR)], rec_v, lsem)
        pltpu.async_copy(alpha_hbm.at[pl.ds(k * CHR, CHR)], al_v, lsem)

    def wait_load(k, rec_v, al_v):
        pltpu.make_async_copy(sd_hbm.at[pl.ds(k * CHR, CHR)], rec_v,
                              lsem).wait()
        pltpu.make_async_copy(alpha_hbm.at[pl.ds(k * CHR, CHR)], al_v,
                              lsem).wait()

    def process(k, rec_v, al_v):
        def vec(j, off_v):
            s = rec_v[j // 8, 0, pl.ds((j % 8) * 16, 16)]
            d = rec_v[j // 8, 1, pl.ds((j % 8) * 16, 16)]
            a = al_v[j // 8, pl.ds((j % 8) * 16, 16)]
            e = jnp.exp(a - mvec)
            dl = d - sb
            mask = dl.astype(jnp.uint32) < jnp.uint32(STRIPE)
            cs = plsc.cumsum(mask.astype(jnp.int32))
            p = off_v + cs - 1
            plsc.store_scatter(sbuf, [p], s, mask=mask)
            plsc.store_scatter(dbuf, [p], dl, mask=mask)
            plsc.store_scatter(ebuf, [p], e, mask=mask)
            return off_v + plsc.all_reduce_population_count(mask)

        off_v = lax.fori_loop(0, CHR * 8, vec,
                              jnp.zeros((16,), jnp.int32), unroll=4)
        off = jnp.max(off_v)
        ng = (off + 15) // 16
        nb = (ng + 3) // 4

        def block(b, _):
            gcnt = jnp.minimum(ng - b * 4, 4)

            def issue(g, _):
                sl = pl.ds((b * 4 + g) * 16, 16)
                rl = pl.ds(g * 16, 16)
                pltpu.async_copy(h_hbm.at[sbuf.at[sl]], rows_v.at[rl],
                                 gsem)
                return 0

            lax.fori_loop(0, gcnt, issue, 0)

            def drain(g, _):
                gg = b * 4 + g
                sl = pl.ds(gg * 16, 16)
                rl = pl.ds(g * 16, 16)
                pltpu.make_async_copy(h_hbm.at[sbuf.at[sl]],
                                      rows_v.at[rl], gsem).wait()
                dv = dbuf[sl]
                ev = ebuf[sl]
                ev = jnp.where(gg * 16 + iota16 < off, ev, 0.0)
                for r in range(16):
                    dls = dv[r]
                    ems = ev[r]
                    for q in range(nq):
                        plsc.addupdate(
                            acc.at[dls, pl.ds(q * 16, 16)],
                            rows_v[g * 16 + r, pl.ds(q * 16, 16)] * ems)
                    plsc.addupdate(accd.at[pl.ds(dls * 16, 16)],
                                   jnp.full((16,), ems, jnp.float32))
                return 0

            lax.fori_loop(0, gcnt, drain, 0)
            return 0

        lax.fori_loop(0, nb, block, 0)

    issue_load(0, rec_a, al_a)

    def body2(k2, _):
        k = 2 * k2
        issue_load(k + 1, rec_b, al_b)
        wait_load(k, rec_a, al_a)
        process(k, rec_a, al_a)

        @pl.when(k2 < NCHUNK // 2 - 1)
        def _():
            issue_load(k + 2, rec_a, al_a)

        wait_load(k + 1, rec_b, al_b)
        process(k + 1, rec_b, al_b)
        return 0

    lax.fori_loop(0, NCHUNK // 2, body2, 0)
    pltpu.sync_copy(acc, numer_hbm.at[pl.ds(sb, STRIPE)])
    pltpu.sync_copy(accd, denp_hbm.at[pl.ds(sb * 16, STRIPE * 16)])


def _sc_scatter(alpha2d, m, sd, h, f):
    fn = pl.kernel(
        functools.partial(_sc_scatter_body, f=f),
        out_type=(jax.ShapeDtypeStruct((N_PAD, f), jnp.float32),
                  jax.ShapeDtypeStruct((N_PAD * 16,), jnp.float32)),
        mesh=_sc_mesh,
        compiler_params=_sc_params,
        scratch_types=[
            pltpu.VMEM((STRIPE, f), jnp.float32),
            pltpu.VMEM((STRIPE * 16,), jnp.float32),
            pltpu.VMEM((CHR, 2, 128), jnp.int32),
            pltpu.VMEM((CHR, 2, 128), jnp.int32),
            pltpu.VMEM((CHR, 128), jnp.float32),
            pltpu.VMEM((CHR, 128), jnp.float32),
            pltpu.VMEM((CHR * 128,), jnp.int32),
            pltpu.VMEM((CHR * 128,), jnp.int32),
            pltpu.VMEM((CHR * 128,), jnp.float32),
            pltpu.VMEM((64, f), jnp.float32),
            pltpu.VMEM((16,), jnp.float32),
            pltpu.SemaphoreType.DMA,
            pltpu.SemaphoreType.DMA,
        ],
    )
    return fn(alpha2d, m, sd, h)


def kernel(x, edge_index, edge_attr, batch, emb,
           W1, a_s1, a_d1, We1, ae1, b1,
           W2, a_s2, a_d2, We2, ae2, b2,
           W3, a_s3, a_d3, We3, ae3, b3):
    src = jnp.concatenate([edge_index[0],
                           jnp.zeros((E_PAD - E,), jnp.int32)])
    dst = jnp.concatenate([edge_index[1],
                           jnp.full((E_PAD - E,), 1 << 30, jnp.int32)])

    v8 = jnp.zeros((DE, 8), jnp.float32)
    v8 = v8.at[:, 0].set(We1 @ ae1).at[:, 1].set(We2 @ ae2)
    v8 = v8.at[:, 2].set(We3 @ ae3)
    ea2 = edge_attr.reshape(E // 8, 128)
    bd = jnp.kron(jnp.eye(8, dtype=jnp.float32), v8)
    t16 = jnp.kron(jnp.ones((8, 1), jnp.float32),
                   jnp.eye(16, dtype=jnp.float32))
    c64, clv = _edge_feat(ea2, bd, t16, v8)
    c8 = c64.reshape(E_PAD, 8)

    W3p = jnp.concatenate([W3, jnp.zeros((H, F3 - C), jnp.float32)], 1)
    b3p = jnp.concatenate([b3, jnp.zeros((F3 - C,), jnp.float32)])
    zc = jnp.zeros((F3 - C,), jnp.float32)
    a_s3p = jnp.concatenate([a_s3, zc])
    a_d3p = jnp.concatenate([a_d3, zc])

    h1, pack = _embed_l1(x, emb, W1, a_s1, a_d1, clv)

    src2d = src.reshape(E_PAD // 128, 128)
    dst2d = dst.reshape(E_PAD // 128, 128)
    sd = jnp.stack([src2d, dst2d], axis=1)

    layer_w = [(W2, a_s2, a_d2, b1, 1, H), (W3p, a_s3p, a_d3p, b2, 2, F3)]
    h, f = h1, H
    for i in range(3):
        c_l = c8[:, i].reshape(E_PAD // 128, 128)
        asc, adc, al = pack[:, 0], pack[:, 1], pack[:, 2:3]
        alpha2d, tmax = _sc_alpha(asc, adc, src2d, dst2d, c_l)
        m = _gmax(tmax, al)
        m16 = m.reshape(16)
        numer, denp = _sc_scatter(alpha2d, m16, sd, h, f)
        denp = denp.reshape(N_PAD, 16)
        if i < 2:
            w_n, as_n, ad_n, b_l, li, f_n = layer_w[i]
            h, pack = _post_layer(numer, denp, al, m, h, b_l,
                                  w_n, as_n, ad_n, clv, li, f_n)
            f = f_n
        else:
            bat_row = batch[None, :]
            out = _final(numer, denp, al, m, h, b3p, bat_row)
    return out
